# Initial kernel scaffold; baseline (speedup 1.0000x reference)
import jax
import jax.numpy as jnp
from jax.experimental import pallas as pl


def kernel(entity_boxes, neighbor_boxes, entity_relations, entity_rel_lens, neighbor_relations, neighbor_rel_lens):
    raise NotImplementedError("write your pallas kernel here")


# speedup vs baseline: 3.8485x; 3.8485x over previous
import functools

import jax
import jax.numpy as jnp
from jax import lax
from jax.experimental import pallas as pl
from jax.experimental.pallas import tpu as pltpu
from jax.experimental.pallas import tpu_sc as plsc

_EULER_GAMMA = 0.5772156649015329
_EPS_VOL = 1e-23
_TINY = 1e-13
_CLAMP = 10.0

_B, _N, _D, _LE, _LN = 128, 32, 512, 64, 64
_NC, _NS, _NL = 2, 16, 16
_BPW = _B // (_NC * _NS)


def _log_vol(l, r):
    return jnp.sum(jnp.log(jax.nn.softplus(r - l - 2.0 * _EULER_GAMMA) + _EPS_VOL), axis=-1)


def _pred_body(e_ref, nb_ref, out_ref):
    el = e_ref[:, 0, :]
    er = e_ref[:, 1, :]
    nl = nb_ref[:, :, 0, :]
    nr = nb_ref[:, :, 1, :]
    el_b = el[:, None, :]
    er_b = er[:, None, :]
    il = jnp.logaddexp(el_b, nl)
    ir = -jnp.logaddexp(-er_b, -nr)
    log_int = _log_vol(il, ir)
    log_ent = _log_vol(el, er)[:, None]
    log_neigh = _log_vol(nl, nr)
    log_sum = jnp.logaddexp(log_ent, log_neigh)
    d = jnp.minimum(log_int - log_sum, -1e-7)
    log_union = log_sum + jnp.log1p(-jnp.exp(d))
    log_pred = log_int - log_union
    out_ref[:, :] = jnp.exp(jnp.clip(log_pred, -_CLAMP, _CLAMP))


def _pred_pallas(entity_boxes, neighbor_boxes):
    bb = 8
    grid = (_B // bb,)
    return pl.pallas_call(
        _pred_body,
        grid=grid,
        in_specs=[
            pl.BlockSpec((bb, 2, _D), lambda i: (i, 0, 0)),
            pl.BlockSpec((bb, _N, 2, _D), lambda i: (i, 0, 0, 0)),
        ],
        out_specs=pl.BlockSpec((bb, _N), lambda i: (i, 0)),
        out_shape=jax.ShapeDtypeStruct((_B, _N), jnp.float32),
    )(entity_boxes, neighbor_boxes)


def _sc_target(ent_rels, ent_lens, neigh_rels, neigh_lens):
    mesh = plsc.VectorSubcoreMesh(core_axis_name="c", subcore_axis_name="s")

    @functools.partial(
        pl.kernel,
        out_type=jax.ShapeDtypeStruct((_B, _N), jnp.float32),
        mesh=mesh,
        compiler_params=pltpu.CompilerParams(needs_layout_passes=False),
        scratch_types=[
            pltpu.VMEM((_BPW, _LE), jnp.int32),
            pltpu.VMEM((_B,), jnp.int32),
            pltpu.VMEM((_BPW, _N, _LN), jnp.int32),
            pltpu.VMEM((_BPW, _N), jnp.int32),
            pltpu.VMEM((512,), jnp.int32),
            pltpu.VMEM((_BPW, _N), jnp.float32),
        ],
    )
    def body(er_hbm, elen_hbm, nr_hbm, nlen_hbm, out_hbm, ev, elv, nv, nlv, hist, outl):
        wid = lax.axis_index("s") * _NC + lax.axis_index("c")
        b0 = wid * _BPW
        pltpu.sync_copy(er_hbm.at[pl.ds(b0, _BPW)], ev)
        pltpu.sync_copy(elen_hbm, elv)
        pltpu.sync_copy(nr_hbm.at[pl.ds(b0, _BPW)], nv)
        pltpu.sync_copy(nlen_hbm.at[pl.ds(b0, _BPW)], nlv)
        lanes = lax.broadcasted_iota(jnp.int32, (_NL,), 0)
        for k in range(512 // _NL):
            hist[pl.ds(k * _NL, _NL)] = jnp.zeros((_NL,), jnp.int32)
        for j in range(_BPW):
            le_vec = plsc.load_gather(elv, [jnp.full((_NL,), b0 + j, jnp.int32)])
            le_f = le_vec.astype(jnp.float32)
            evs, ems, ecnts, elasts = [], [], [], []
            for k in range(_LE // _NL):
                v = ev[j, pl.ds(k * _NL, _NL)]
                m = (lanes + k * _NL) < le_vec
                cnt, last = plsc.scan_count(v, m)
                plsc.addupdate_scatter(hist, [v], cnt, mask=last)
                evs.append(v); ems.append(m); ecnts.append(cnt); elasts.append(last)
            for h in range(2):
                def nbody(i, out_vec, h=h):
                    n = h * _NL + i
                    jf = jnp.full((_NL,), j, jnp.int32)
                    nf = jnp.full((_NL,), n, jnp.int32)
                    ln_vec = plsc.load_gather(nlv, [jf, nf])
                    nvs, nms, ncnts, nlasts = [], [], [], []
                    for k in range(_LN // _NL):
                        vv = plsc.load_gather(nv, [jf, nf, lanes + k * _NL])
                        mm = (lanes + k * _NL) < ln_vec
                        cnt, last = plsc.scan_count(vv, mm)
                        plsc.addupdate_scatter(hist, [vv], cnt, mask=last)
                        nvs.append(vv); nms.append(mm); ncnts.append(cnt); nlasts.append(last)
                    acc = jnp.zeros((_NL,), jnp.float32)
                    for v, m in zip(evs + nvs, ems + nms):
                        occ = plsc.load_gather(hist, [v], mask=m)
                        occ_f = occ.astype(jnp.float32)
                        acc = acc + jnp.where(m & (occ >= 2), 1.0 / occ_f, 0.0)
                    for k in range(_LN // _NL):
                        plsc.addupdate_scatter(hist, [nvs[k]], -ncnts[k], mask=nlasts[k])
                    inter = jnp.full((_NL,), jnp.sum(acc), jnp.float32)
                    t_vec = inter / (le_f + ln_vec.astype(jnp.float32) + _TINY)
                    return jnp.where(lanes == i, t_vec, out_vec)
                out_vec = lax.fori_loop(0, _NL, nbody, jnp.zeros((_NL,), jnp.float32))
                outl[j, pl.ds(h * _NL, _NL)] = out_vec
            for k in range(_LE // _NL):
                plsc.addupdate_scatter(hist, [evs[k]], -ecnts[k], mask=elasts[k])
        pltpu.sync_copy(outl, out_hbm.at[pl.ds(b0, _BPW)])

    return body(ent_rels, ent_lens, neigh_rels, neigh_lens)


def _combine_body(pred_ref, tgt_ref, elen_ref, nlen_ref, out_ref):
    pred = pred_ref[...]
    tgt = tgt_ref[...]
    el = elen_ref[...]
    nl = nlen_ref[...]
    loss = (pred - tgt) ** 2
    mask = (nl >= 1) & (el >= 1)
    loss = jnp.where(mask, loss, 0.0)
    out_ref[0, 0] = jnp.sum(loss) / float(_B * _N)


def _combine_pallas(pred, tgt, ent_lens, neigh_lens):
    return pl.pallas_call(
        _combine_body,
        in_specs=[
            pl.BlockSpec(memory_space=pltpu.VMEM),
            pl.BlockSpec(memory_space=pltpu.VMEM),
            pl.BlockSpec(memory_space=pltpu.VMEM),
            pl.BlockSpec(memory_space=pltpu.VMEM),
        ],
        out_specs=pl.BlockSpec(memory_space=pltpu.SMEM),
        out_shape=jax.ShapeDtypeStruct((1, 1), jnp.float32),
    )(pred, tgt, ent_lens.reshape(_B, 1), neigh_lens)


def kernel(entity_boxes, neighbor_boxes, entity_relations, entity_rel_lens,
           neighbor_relations, neighbor_rel_lens):
    pred = _pred_pallas(entity_boxes, neighbor_boxes)
    tgt = _sc_target(entity_relations, entity_rel_lens,
                     neighbor_relations, neighbor_rel_lens)
    out = _combine_pallas(pred, tgt, entity_rel_lens, neighbor_rel_lens)
    return out[0, 0]


# speedup vs baseline: 4.5793x; 1.1899x over previous
import functools

import jax
import jax.numpy as jnp
from jax import lax
from jax.experimental import pallas as pl
from jax.experimental.pallas import tpu as pltpu
from jax.experimental.pallas import tpu_sc as plsc

_EULER_GAMMA = 0.5772156649015329
_EPS_VOL = 1e-23
_TINY = 1e-13
_CLAMP = 10.0

_B, _N, _D, _LE, _LN = 128, 32, 512, 64, 64
_NC, _NS, _NL = 2, 16, 16
_BPW = _B // (_NC * _NS)


def _l1pe(x):
    return jnp.log(1.0 + jnp.exp(-jnp.abs(x)))


def _lae(a, b):
    return jnp.maximum(a, b) + _l1pe(a - b)


def _log_vol_terms(x):
    sp = jnp.maximum(x, 0.0) + _l1pe(x)
    return jnp.log(sp + _EPS_VOL)


def _log_vol(l, r):
    return jnp.sum(_log_vol_terms(r - l - 2.0 * _EULER_GAMMA), axis=-1)


def _pred_body(e_ref, nb_ref, out_ref):
    el = e_ref[:, 0, :]
    er = e_ref[:, 1, :]
    nl = nb_ref[:, :, 0, :]
    nr = nb_ref[:, :, 1, :]
    el_b = el[:, None, :]
    er_b = er[:, None, :]
    il = _lae(el_b, nl)
    ir = jnp.minimum(er_b, nr) - _l1pe(er_b - nr)
    log_int = _log_vol(il, ir)
    log_ent = _log_vol(el, er)[:, None]
    log_neigh = _log_vol(nl, nr)
    log_sum = _lae(log_ent, log_neigh)
    d = jnp.minimum(log_int - log_sum, -1e-7)
    log_union = log_sum + jnp.log1p(-jnp.exp(d))
    log_pred = log_int - log_union
    out_ref[:, :] = jnp.exp(jnp.clip(log_pred, -_CLAMP, _CLAMP))


def _pred_pallas(entity_boxes, neighbor_boxes):
    bb = 8
    grid = (_B // bb,)
    return pl.pallas_call(
        _pred_body,
        grid=grid,
        in_specs=[
            pl.BlockSpec((bb, 2, _D), lambda i: (i, 0, 0)),
            pl.BlockSpec((bb, _N, 2, _D), lambda i: (i, 0, 0, 0)),
        ],
        out_specs=pl.BlockSpec((bb, _N), lambda i: (i, 0)),
        out_shape=jax.ShapeDtypeStruct((_B, _N), jnp.float32),
    )(entity_boxes, neighbor_boxes)


def _sc_target(ent_rels, ent_lens, neigh_rels, neigh_lens):
    mesh = plsc.VectorSubcoreMesh(core_axis_name="c", subcore_axis_name="s")

    @functools.partial(
        pl.kernel,
        out_type=jax.ShapeDtypeStruct((_B, _N), jnp.float32),
        mesh=mesh,
        compiler_params=pltpu.CompilerParams(needs_layout_passes=False),
        scratch_types=[
            pltpu.VMEM((_BPW, _LE), jnp.int32),
            pltpu.VMEM((_B,), jnp.int32),
            pltpu.VMEM((_BPW, _N, _LN), jnp.int32),
            pltpu.VMEM((_BPW, _N), jnp.int32),
            pltpu.VMEM((512,), jnp.int32),
            pltpu.VMEM((_BPW, _N), jnp.float32),
        ],
    )
    def body(er_hbm, elen_hbm, nr_hbm, nlen_hbm, out_hbm, ev, elv, nv, nlv, hist, outl):
        wid = lax.axis_index("s") * _NC + lax.axis_index("c")
        b0 = wid * _BPW
        pltpu.sync_copy(er_hbm.at[pl.ds(b0, _BPW)], ev)
        pltpu.sync_copy(elen_hbm, elv)
        pltpu.sync_copy(nr_hbm.at[pl.ds(b0, _BPW)], nv)
        pltpu.sync_copy(nlen_hbm.at[pl.ds(b0, _BPW)], nlv)
        lanes = lax.broadcasted_iota(jnp.int32, (_NL,), 0)
        for k in range(512 // _NL):
            hist[pl.ds(k * _NL, _NL)] = jnp.zeros((_NL,), jnp.int32)
        for j in range(_BPW):
            le_vec = plsc.load_gather(elv, [jnp.full((_NL,), b0 + j, jnp.int32)])
            le_f = le_vec.astype(jnp.float32)
            evs, ems, ecnts, elasts = [], [], [], []
            for k in range(_LE // _NL):
                v = ev[j, pl.ds(k * _NL, _NL)]
                m = (lanes + k * _NL) < le_vec
                cnt, last = plsc.scan_count(v, m)
                plsc.addupdate_scatter(hist, [v], cnt, mask=last)
                evs.append(v); ems.append(m); ecnts.append(cnt); elasts.append(last)
            for h in range(2):
                def nbody(i, out_vec, h=h):
                    n = h * _NL + i
                    jf = jnp.full((_NL,), j, jnp.int32)
                    nf = jnp.full((_NL,), n, jnp.int32)
                    ln_vec = plsc.load_gather(nlv, [jf, nf])
                    nvs, nms, ncnts, nlasts = [], [], [], []
                    for k in range(_LN // _NL):
                        vv = plsc.load_gather(nv, [jf, nf, lanes + k * _NL])
                        mm = (lanes + k * _NL) < ln_vec
                        cnt, last = plsc.scan_count(vv, mm)
                        plsc.addupdate_scatter(hist, [vv], cnt, mask=last)
                        nvs.append(vv); nms.append(mm); ncnts.append(cnt); nlasts.append(last)
                    acc = jnp.zeros((_NL,), jnp.float32)
                    for v, m in zip(evs + nvs, ems + nms):
                        occ = plsc.load_gather(hist, [v], mask=m)
                        occ_f = occ.astype(jnp.float32)
                        acc = acc + jnp.where(m & (occ >= 2), 1.0 / occ_f, 0.0)
                    for k in range(_LN // _NL):
                        plsc.addupdate_scatter(hist, [nvs[k]], -ncnts[k], mask=nlasts[k])
                    inter = jnp.full((_NL,), jnp.sum(acc), jnp.float32)
                    t_vec = inter / (le_f + ln_vec.astype(jnp.float32) + _TINY)
                    return jnp.where(lanes == i, t_vec, out_vec)
                out_vec = lax.fori_loop(0, _NL, nbody, jnp.zeros((_NL,), jnp.float32))
                outl[j, pl.ds(h * _NL, _NL)] = out_vec
            for k in range(_LE // _NL):
                plsc.addupdate_scatter(hist, [evs[k]], -ecnts[k], mask=elasts[k])
        pltpu.sync_copy(outl, out_hbm.at[pl.ds(b0, _BPW)])

    return body(ent_rels, ent_lens, neigh_rels, neigh_lens)


def _combine_body(pred_ref, tgt_ref, elen_ref, nlen_ref, out_ref):
    pred = pred_ref[...]
    tgt = tgt_ref[...]
    el = elen_ref[...]
    nl = nlen_ref[...]
    loss = (pred - tgt) ** 2
    mask = (nl >= 1) & (el >= 1)
    loss = jnp.where(mask, loss, 0.0)
    out_ref[0, 0] = jnp.sum(loss) / float(_B * _N)


def _combine_pallas(pred, tgt, ent_lens, neigh_lens):
    return pl.pallas_call(
        _combine_body,
        in_specs=[
            pl.BlockSpec(memory_space=pltpu.VMEM),
            pl.BlockSpec(memory_space=pltpu.VMEM),
            pl.BlockSpec(memory_space=pltpu.VMEM),
            pl.BlockSpec(memory_space=pltpu.VMEM),
        ],
        out_specs=pl.BlockSpec(memory_space=pltpu.SMEM),
        out_shape=jax.ShapeDtypeStruct((1, 1), jnp.float32),
    )(pred, tgt, ent_lens.reshape(_B, 1), neigh_lens)


def kernel(entity_boxes, neighbor_boxes, entity_relations, entity_rel_lens,
           neighbor_relations, neighbor_rel_lens):
    pred = _pred_pallas(entity_boxes, neighbor_boxes)
    tgt = _sc_target(entity_relations, entity_rel_lens,
                     neighbor_relations, neighbor_rel_lens)
    out = _combine_pallas(pred, tgt, entity_rel_lens, neighbor_rel_lens)
    return out[0, 0]


# speedup vs baseline: 4.5851x; 1.0013x over previous
import functools

import jax
import jax.numpy as jnp
from jax import lax
from jax.experimental import pallas as pl
from jax.experimental.pallas import tpu as pltpu
from jax.experimental.pallas import tpu_sc as plsc

_EULER_GAMMA = 0.5772156649015329
_EPS_VOL = 1e-23
_TINY = 1e-13
_CLAMP = 10.0

_B, _N, _D, _LE, _LN = 128, 32, 512, 64, 64
_NC, _NS, _NL = 2, 16, 16
_BPW = _B // (_NC * _NS)


def _l1pe(x):
    return jnp.log(1.0 + jnp.exp(-jnp.abs(x)))


def _lae(a, b):
    return jnp.maximum(a, b) + _l1pe(a - b)


def _log_vol_terms(x):
    sp = jnp.maximum(x, 0.0) + _l1pe(x)
    return jnp.log(sp + _EPS_VOL)


def _log_vol(l, r):
    return jnp.sum(_log_vol_terms(r - l - 2.0 * _EULER_GAMMA), axis=-1)


def _pred_body(e_ref, nb_ref, out_ref):
    el = e_ref[:, 0, :]
    er = e_ref[:, 1, :]
    nl = nb_ref[:, :, 0, :]
    nr = nb_ref[:, :, 1, :]
    el_b = el[:, None, :]
    er_b = er[:, None, :]
    il = _lae(el_b, nl)
    ir = jnp.minimum(er_b, nr) - _l1pe(er_b - nr)
    log_int = _log_vol(il, ir)
    log_ent = _log_vol(el, er)[:, None]
    log_neigh = _log_vol(nl, nr)
    log_sum = _lae(log_ent, log_neigh)
    d = jnp.minimum(log_int - log_sum, -1e-7)
    log_union = log_sum + jnp.log1p(-jnp.exp(d))
    log_pred = log_int - log_union
    out_ref[:, :] = jnp.exp(jnp.clip(log_pred, -_CLAMP, _CLAMP))


def _pred_pallas(entity_boxes, neighbor_boxes):
    bb = 8
    grid = (_B // bb,)
    return pl.pallas_call(
        _pred_body,
        grid=grid,
        in_specs=[
            pl.BlockSpec((bb, 2, _D), lambda i: (i, 0, 0)),
            pl.BlockSpec((bb, _N, 2, _D), lambda i: (i, 0, 0, 0)),
        ],
        out_specs=pl.BlockSpec((bb, _N), lambda i: (i, 0)),
        out_shape=jax.ShapeDtypeStruct((_B, _N), jnp.float32),
    )(entity_boxes, neighbor_boxes)


def _sc_target(ent_rels, ent_lens, neigh_rels, neigh_lens):
    mesh = plsc.VectorSubcoreMesh(core_axis_name="c", subcore_axis_name="s")

    @functools.partial(
        pl.kernel,
        out_type=jax.ShapeDtypeStruct((_B, _N), jnp.float32),
        mesh=mesh,
        compiler_params=pltpu.CompilerParams(needs_layout_passes=False),
        scratch_types=[
            pltpu.VMEM((_BPW, _LE), jnp.int32),
            pltpu.VMEM((_B,), jnp.int32),
            pltpu.VMEM((_BPW, _N, _LN), jnp.int32),
            pltpu.VMEM((_BPW, _N), jnp.int32),
            pltpu.VMEM((512,), jnp.int32),
            pltpu.VMEM((_BPW, _N), jnp.float32),
        ],
    )
    def body(er_hbm, elen_hbm, nr_hbm, nlen_hbm, out_hbm, ev, elv, nv, nlv, hist, outl):
        wid = lax.axis_index("s") * _NC + lax.axis_index("c")
        b0 = wid * _BPW
        pltpu.sync_copy(er_hbm.at[pl.ds(b0, _BPW)], ev)
        pltpu.sync_copy(elen_hbm, elv)
        pltpu.sync_copy(nr_hbm.at[pl.ds(b0, _BPW)], nv)
        pltpu.sync_copy(nlen_hbm.at[pl.ds(b0, _BPW)], nlv)
        lanes = lax.broadcasted_iota(jnp.int32, (_NL,), 0)
        for k in range(512 // _NL):
            hist[pl.ds(k * _NL, _NL)] = jnp.zeros((_NL,), jnp.int32)
        for j in range(_BPW):
            le_vec = plsc.load_gather(elv, [jnp.full((_NL,), b0 + j, jnp.int32)])
            le_f = le_vec.astype(jnp.float32)
            evs, ems, ecnts, elasts = [], [], [], []
            for k in range(_LE // _NL):
                v = ev[j, pl.ds(k * _NL, _NL)]
                m = (lanes + k * _NL) < le_vec
                cnt, last = plsc.scan_count(v, m)
                plsc.addupdate_scatter(hist, [v], cnt, mask=last)
                evs.append(v); ems.append(m); ecnts.append(cnt); elasts.append(last)
            for h in range(2):
                def nbody(i, out_vec, h=h):
                    n = h * _NL + i
                    jf = jnp.full((_NL,), j, jnp.int32)
                    nf = jnp.full((_NL,), n, jnp.int32)
                    ln_vec = plsc.load_gather(nlv, [jf, nf])
                    nvs, nms, ncnts, nlasts = [], [], [], []
                    for k in range(_LN // _NL):
                        vv = plsc.load_gather(nv, [jf, nf, lanes + k * _NL])
                        mm = (lanes + k * _NL) < ln_vec
                        cnt, last = plsc.scan_count(vv, mm)
                        plsc.addupdate_scatter(hist, [vv], cnt, mask=last)
                        nvs.append(vv); nms.append(mm); ncnts.append(cnt); nlasts.append(last)
                    acc = jnp.zeros((_NL,), jnp.float32)
                    for v, m in zip(evs + nvs, ems + nms):
                        occ = plsc.load_gather(hist, [v], mask=m)
                        occ_f = occ.astype(jnp.float32)
                        acc = acc + jnp.where(m & (occ >= 2), 1.0 / occ_f, 0.0)
                    for k in range(_LN // _NL):
                        plsc.addupdate_scatter(hist, [nvs[k]], -ncnts[k], mask=nlasts[k])
                    inter = jnp.full((_NL,), jnp.sum(acc), jnp.float32)
                    t_vec = inter / (le_f + ln_vec.astype(jnp.float32) + _TINY)
                    return jnp.where(lanes == i, t_vec, out_vec)
                out_vec = lax.fori_loop(0, _NL, nbody, jnp.zeros((_NL,), jnp.float32))
                outl[j, pl.ds(h * _NL, _NL)] = out_vec
            for k in range(_LE // _NL):
                plsc.addupdate_scatter(hist, [evs[k]], -ecnts[k], mask=elasts[k])
        pltpu.sync_copy(outl, out_hbm.at[pl.ds(b0, _BPW)])

    return body(ent_rels, ent_lens, neigh_rels, neigh_lens)


def _combine_body(pred_ref, tgt_ref, elen_ref, nlen_ref, out_ref):
    pred = pred_ref[...]
    tgt = tgt_ref[...]
    el = elen_ref[...]
    nl = nlen_ref[...]
    loss = (pred - tgt) ** 2
    mask = (nl >= 1) & (el >= 1)
    loss = jnp.where(mask, loss, 0.0)
    out_ref[0, 0] = jnp.sum(loss) / float(_B * _N)


def _combine_pallas(pred, tgt, ent_lens, neigh_lens):
    return pl.pallas_call(
        _combine_body,
        in_specs=[
            pl.BlockSpec(memory_space=pltpu.VMEM),
            pl.BlockSpec(memory_space=pltpu.VMEM),
            pl.BlockSpec(memory_space=pltpu.VMEM),
            pl.BlockSpec(memory_space=pltpu.VMEM),
        ],
        out_specs=pl.BlockSpec(memory_space=pltpu.SMEM),
        out_shape=jax.ShapeDtypeStruct((1, 1), jnp.float32),
    )(pred, tgt, ent_lens.reshape(_B, 1), neigh_lens)


def kernel(entity_boxes, neighbor_boxes, entity_relations, entity_rel_lens,
           neighbor_relations, neighbor_rel_lens):
    tgt = _sc_target(entity_relations, entity_rel_lens,
                     neighbor_relations, neighbor_rel_lens)
    pred = _pred_pallas(entity_boxes, neighbor_boxes)
    out = _combine_pallas(pred, tgt, entity_rel_lens, neighbor_rel_lens)
    return out[0, 0]
